# baseline (speedup 1.0000x reference)
import functools

import jax
import jax.numpy as jnp
from jax.experimental import pallas as pl
from jax.experimental.pallas import tpu as pltpu


def _merge3(l1, l2, l3, r1, r2, r3):
    y1 = jnp.minimum(l1, r1)
    x2 = jnp.maximum(l2, r2)
    c1 = jnp.maximum(l1, r1)
    c2 = jnp.maximum(y1, x2)
    c3 = jnp.maximum(jnp.minimum(y1, x2), jnp.maximum(l3, r3))
    return c1, c2, c3


def _fold_triple(c1, c2, c3, w, stop):
    while w > stop:
        w //= 2
        c1, c2, c3 = _merge3(
            c1[:, :w], c2[:, :w], c3[:, :w],
            c1[:, w:2 * w], c2[:, w:2 * w], c3[:, w:2 * w],
        )
    return c1, c2, c3


def _body(t1_ref, t2_ref, out_ref, t1n_ref, a1, a2, a3, *, tm, tn, nb, n_valid_last):
    n = pl.program_id(1)

    @pl.when(n == 0)
    def _init():
        t1 = t1_ref[...]
        t1n_ref[...] = t1 * jax.lax.rsqrt(
            jnp.sum(t1 * t1, axis=1, keepdims=True)
        )
        neg = jnp.full((tm, 128), -jnp.inf, jnp.float32)
        a1[...] = neg
        a2[...] = neg
        a3[...] = neg

    def _t2n_full():
        t2 = t2_ref[...]
        return t2 * jax.lax.rsqrt(jnp.sum(t2 * t2, axis=1, keepdims=True))

    def _dot(t2n_part):
        return jax.lax.dot_general(
            t1n_ref[...],
            t2n_part,
            (((1,), (1,)), ((), ())),
            preferred_element_type=jnp.float32,
        )

    def _parts_pairs(num_parts, rem):
        part = 256
        t2b = t2_ref[...]
        pairs = []
        for j in range(num_parts):
            t2p = t2b[j * part:(j + 1) * part, :]
            t2pn = t2p * jax.lax.rsqrt(
                jnp.sum(t2p * t2p, axis=1, keepdims=True)
            )
            sp = _dot(t2pn)
            if rem and j == num_parts - 1:
                col = jax.lax.broadcasted_iota(jnp.int32, (tm, part), 1)
                sp = jnp.where(col < rem, sp, -jnp.inf)
            pairs.append((jnp.maximum(sp[:, :128], sp[:, 128:]),
                          jnp.minimum(sp[:, :128], sp[:, 128:])))
        return pairs

    def _pairs_into_carry(pairs):
        tris = []
        for k in range(0, len(pairs) - 1, 2):
            (h1, l1), (h2, l2) = pairs[k], pairs[k + 1]
            y1 = jnp.minimum(h1, h2)
            x2 = jnp.maximum(l1, l2)
            tris.append((jnp.maximum(h1, h2), jnp.maximum(y1, x2),
                         jnp.minimum(y1, x2)))
        if len(pairs) % 2:
            h, l = pairs[-1]
            tris.append((h, l, jnp.full((tm, 128), -jnp.inf, jnp.float32)))
        while len(tris) > 1:
            nxt = [_merge3(*tris[k], *tris[k + 1])
                   for k in range(0, len(tris) - 1, 2)]
            if len(tris) % 2:
                nxt.append(tris[-1])
            tris = nxt
        b1, b2, b3 = _merge3(a1[...], a2[...], a3[...], *tris[0])
        a1[...] = b1
        a2[...] = b2
        a3[...] = b3

    def _accumulate(s):
        if tn == 128:
            b1 = jnp.maximum(a1[...], s)
            m1 = jnp.minimum(a1[...], s)
            b2 = jnp.maximum(a2[...], m1)
            m2 = jnp.minimum(a2[...], m1)
            b3 = jnp.maximum(a3[...], m2)
        else:
            w = tn // 2
            hi = jnp.maximum(s[:, :w], s[:, w:])
            lo = jnp.minimum(s[:, :w], s[:, w:])
            if w == 128:
                y1 = jnp.minimum(a1[...], hi)
                x2 = jnp.maximum(a2[...], lo)
                b1 = jnp.maximum(a1[...], hi)
                b2 = jnp.maximum(y1, x2)
                b3 = jnp.maximum(jnp.minimum(y1, x2), a3[...])
            else:
                w //= 2
                h1, l1 = hi[:, :w], lo[:, :w]
                h2, l2 = hi[:, w:], lo[:, w:]
                y1 = jnp.minimum(h1, h2)
                x2 = jnp.maximum(l1, l2)
                c1 = jnp.maximum(h1, h2)
                c2 = jnp.maximum(y1, x2)
                c3 = jnp.minimum(y1, x2)
                c1, c2, c3 = _fold_triple(c1, c2, c3, w, 128)
                b1, b2, b3 = _merge3(a1[...], a2[...], a3[...], c1, c2, c3)
        a1[...] = b1
        a2[...] = b2
        a3[...] = b3

    @pl.when(n < nb - 1)
    def _interior():
        if tn >= 512:
            _pairs_into_carry(_parts_pairs(tn // 256, 0))
        else:
            _accumulate(_dot(_t2n_full()))

    @pl.when(n == nb - 1)
    def _last():
        if tn >= 512:
            full, rem = divmod(n_valid_last, 256)
            _pairs_into_carry(_parts_pairs(full + (1 if rem else 0), rem))
        elif n_valid_last < tn:
            col = jax.lax.broadcasted_iota(jnp.int32, (tm, tn), 1)
            _accumulate(jnp.where(col < n_valid_last, _dot(_t2n_full()), -jnp.inf))
        else:
            _accumulate(_dot(_t2n_full()))

        c1, c2, c3 = _fold_triple(a1[...], a2[...], a3[...], 128, 1)
        out_ref[...] = (c1 + c2 + c3) * (1.0 / 3.0)


def _cosine_top3_mean(tensor_1, tensor_2, tm, tn, interpret=False):
    m, k = tensor_1.shape
    n = tensor_2.shape[0]
    nb = pl.cdiv(n, tn)
    n_valid_last = n - (nb - 1) * tn

    out = pl.pallas_call(
        functools.partial(_body, tm=tm, tn=tn, nb=nb, n_valid_last=n_valid_last),
        grid=(m // tm, nb),
        in_specs=[
            pl.BlockSpec((tm, k), lambda i, j: (i, 0)),
            pl.BlockSpec((tn, k), lambda i, j: (j, 0)),
        ],
        out_specs=pl.BlockSpec((tm, 1), lambda i, j: (i, 0)),
        out_shape=jax.ShapeDtypeStruct((m, 1), jnp.float32),
        scratch_shapes=[
            pltpu.VMEM((tm, k), jnp.float32),
            pltpu.VMEM((tm, 128), jnp.float32),
            pltpu.VMEM((tm, 128), jnp.float32),
            pltpu.VMEM((tm, 128), jnp.float32),
        ],
        compiler_params=pltpu.CompilerParams(
            dimension_semantics=("arbitrary", "arbitrary"),
        ),
        interpret=interpret,
    )(tensor_1, tensor_2)
    return jnp.reshape(out, (m,))


def kernel(tensor_1, tensor_2):
    return _cosine_top3_mean(tensor_1, tensor_2, tm=4096, tn=8192)


# speedup vs baseline: 1.7559x; 1.7559x over previous
import functools

import jax
import jax.numpy as jnp
from jax.experimental import pallas as pl
from jax.experimental.pallas import tpu as pltpu


def _merge3(l1, l2, l3, r1, r2, r3):
    y1 = jnp.minimum(l1, r1)
    x2 = jnp.maximum(l2, r2)
    c1 = jnp.maximum(l1, r1)
    c2 = jnp.maximum(y1, x2)
    c3 = jnp.maximum(jnp.minimum(y1, x2), jnp.maximum(l3, r3))
    return c1, c2, c3


def _fold_triple(c1, c2, c3, w, stop):
    while w > stop:
        w //= 2
        c1, c2, c3 = _merge3(
            c1[:, :w], c2[:, :w], c3[:, :w],
            c1[:, w:2 * w], c2[:, w:2 * w], c3[:, w:2 * w],
        )
    return c1, c2, c3


def _body(t1_ref, t2_ref, out_ref, t1n_ref, a1, a2, a3, *, tm, tn, nb, n_valid_last):
    n = pl.program_id(1)

    @pl.when(n == 0)
    def _init():
        t1 = t1_ref[...]
        t1n_ref[...] = t1 * jax.lax.rsqrt(
            jnp.sum(t1 * t1, axis=1, keepdims=True)
        )
        neg = jnp.full((tm, 128), -jnp.inf, jnp.float32)
        a1[...] = neg
        a2[...] = neg
        a3[...] = neg

    def _t2n_full():
        t2 = t2_ref[...]
        return t2 * jax.lax.rsqrt(jnp.sum(t2 * t2, axis=1, keepdims=True))

    def _dot(t2n_part):
        return jax.lax.dot_general(
            t1n_ref[...],
            t2n_part,
            (((1,), (1,)), ((), ())),
            preferred_element_type=jnp.float32,
        )

    def _parts_into_carry(num_parts, rem):
        part = 256
        t2b = t2_ref[...]

        def mk_pair(j):
            t2p = t2b[j * part:(j + 1) * part, :]
            t2pn = t2p * jax.lax.rsqrt(
                jnp.sum(t2p * t2p, axis=1, keepdims=True)
            )
            sp = _dot(t2pn)
            if rem and j == num_parts - 1:
                col = jax.lax.broadcasted_iota(jnp.int32, (tm, part), 1)
                sp = jnp.where(col < rem, sp, -jnp.inf)
            return (jnp.maximum(sp[:, :128], sp[:, 128:]),
                    jnp.minimum(sp[:, :128], sp[:, 128:]))

        stack = []
        idx = 0
        while idx < num_parts:
            if idx + 1 < num_parts:
                h1, l1 = mk_pair(idx)
                h2, l2 = mk_pair(idx + 1)
                idx += 2
                y1 = jnp.minimum(h1, h2)
                x2 = jnp.maximum(l1, l2)
                tri = (jnp.maximum(h1, h2), jnp.maximum(y1, x2),
                       jnp.minimum(y1, x2))
            else:
                h, l = mk_pair(idx)
                idx += 1
                tri = (h, l, jnp.full((tm, 128), -jnp.inf, jnp.float32))
            rank = 0
            while stack and stack[-1][0] == rank:
                _, other = stack.pop()
                tri = _merge3(*other, *tri)
                rank += 1
            stack.append((rank, tri))
        tri = stack.pop()[1]
        while stack:
            tri = _merge3(*stack.pop()[1], *tri)
        b1, b2, b3 = _merge3(a1[...], a2[...], a3[...], *tri)
        a1[...] = b1
        a2[...] = b2
        a3[...] = b3

    def _accumulate(s):
        if tn == 128:
            b1 = jnp.maximum(a1[...], s)
            m1 = jnp.minimum(a1[...], s)
            b2 = jnp.maximum(a2[...], m1)
            m2 = jnp.minimum(a2[...], m1)
            b3 = jnp.maximum(a3[...], m2)
        else:
            w = tn // 2
            hi = jnp.maximum(s[:, :w], s[:, w:])
            lo = jnp.minimum(s[:, :w], s[:, w:])
            if w == 128:
                y1 = jnp.minimum(a1[...], hi)
                x2 = jnp.maximum(a2[...], lo)
                b1 = jnp.maximum(a1[...], hi)
                b2 = jnp.maximum(y1, x2)
                b3 = jnp.maximum(jnp.minimum(y1, x2), a3[...])
            else:
                w //= 2
                h1, l1 = hi[:, :w], lo[:, :w]
                h2, l2 = hi[:, w:], lo[:, w:]
                y1 = jnp.minimum(h1, h2)
                x2 = jnp.maximum(l1, l2)
                c1 = jnp.maximum(h1, h2)
                c2 = jnp.maximum(y1, x2)
                c3 = jnp.minimum(y1, x2)
                c1, c2, c3 = _fold_triple(c1, c2, c3, w, 128)
                b1, b2, b3 = _merge3(a1[...], a2[...], a3[...], c1, c2, c3)
        a1[...] = b1
        a2[...] = b2
        a3[...] = b3

    @pl.when(n < nb - 1)
    def _interior():
        if tn >= 512:
            _parts_into_carry(tn // 256, 0)
        else:
            _accumulate(_dot(_t2n_full()))

    @pl.when(n == nb - 1)
    def _last():
        if tn >= 512:
            full, rem = divmod(n_valid_last, 256)
            _parts_into_carry(full + (1 if rem else 0), rem)
        elif n_valid_last < tn:
            col = jax.lax.broadcasted_iota(jnp.int32, (tm, tn), 1)
            _accumulate(jnp.where(col < n_valid_last, _dot(_t2n_full()), -jnp.inf))
        else:
            _accumulate(_dot(_t2n_full()))

        c1, c2, c3 = _fold_triple(a1[...], a2[...], a3[...], 128, 1)
        out_ref[...] = (c1 + c2 + c3) * (1.0 / 3.0)


def _cosine_top3_mean(tensor_1, tensor_2, tm, tn, interpret=False):
    m, k = tensor_1.shape
    n = tensor_2.shape[0]
    nb = pl.cdiv(n, tn)
    n_valid_last = n - (nb - 1) * tn

    out = pl.pallas_call(
        functools.partial(_body, tm=tm, tn=tn, nb=nb, n_valid_last=n_valid_last),
        grid=(m // tm, nb),
        in_specs=[
            pl.BlockSpec((tm, k), lambda i, j: (i, 0)),
            pl.BlockSpec((tn, k), lambda i, j: (j, 0)),
        ],
        out_specs=pl.BlockSpec((tm, 1), lambda i, j: (i, 0)),
        out_shape=jax.ShapeDtypeStruct((m, 1), jnp.float32),
        scratch_shapes=[
            pltpu.VMEM((tm, k), jnp.float32),
            pltpu.VMEM((tm, 128), jnp.float32),
            pltpu.VMEM((tm, 128), jnp.float32),
            pltpu.VMEM((tm, 128), jnp.float32),
        ],
        compiler_params=pltpu.CompilerParams(
            dimension_semantics=("arbitrary", "arbitrary"),
        ),
        interpret=interpret,
    )(tensor_1, tensor_2)
    return jnp.reshape(out, (m,))


def kernel(tensor_1, tensor_2):
    return _cosine_top3_mean(tensor_1, tensor_2, tm=4096, tn=4096)
